# Initial kernel scaffold; baseline (speedup 1.0000x reference)
import jax
import jax.numpy as jnp
from jax.experimental import pallas as pl


def kernel(x, edge_index, W1, b1, ln1_w, ln1_b, W2, b2, ln2_w, ln2_b):
    raise NotImplementedError("write your pallas kernel here")


# speedup vs baseline: 12.5605x; 12.5605x over previous
import functools

import jax
import jax.numpy as jnp
from jax import lax
from jax.experimental import pallas as pl
from jax.experimental.pallas import tpu as pltpu
from jax.experimental.pallas import tpu_sc as plsc

_NC = 2
_NS = 16
_CH = 80
_ZR = 128
_NPAD = 10240


def _degree_body(epw, nchunk, rpt, dst_hbm, ones_hbm, zeros_hbm, out_hbm,
                 deg_sp, idx_v, ones_v, zb):
    c = lax.axis_index("c")
    s = lax.axis_index("s")
    wid = c * _NS + s
    pltpu.sync_copy(ones_hbm, ones_v)
    pltpu.sync_copy(zeros_hbm, zb)
    nz = rpt // _ZR
    for k in range(nz):
        pltpu.sync_copy(zb, deg_sp.at[pl.ds(s * rpt + k * _ZR, _ZR)])
    plsc.subcore_barrier()

    def body(i, carry):
        off = wid * epw + i * _CH
        pltpu.sync_copy(dst_hbm.at[pl.ds(off, _CH)], idx_v)
        pltpu.sync_copy(ones_v, deg_sp.at[idx_v], add=True)
        return carry

    lax.fori_loop(0, nchunk, body, 0)
    plsc.subcore_barrier()
    for k in range(nz):
        r0 = s * rpt + k * _ZR
        pltpu.sync_copy(deg_sp.at[pl.ds(r0, _ZR)], zb)
        pltpu.sync_copy(zb, out_hbm.at[c, pl.ds(r0, _ZR)])


def _sc_degree(dst, n):
    del n
    e = dst.shape[0]
    epw = e // (_NC * _NS)
    nchunk = epw // _CH
    rpt = _NPAD // _NS
    d = 128
    ones = jnp.ones((_CH, d), jnp.float32)
    zeros = jnp.zeros((_ZR, d), jnp.float32)
    mesh = plsc.VectorSubcoreMesh(core_axis_name="c", subcore_axis_name="s",
                                  num_cores=_NC, num_subcores=_NS)
    run = pl.kernel(
        functools.partial(_degree_body, epw, nchunk, rpt),
        out_type=jax.ShapeDtypeStruct((_NC, _NPAD, d), jnp.float32),
        mesh=mesh,
        scratch_types=[
            pltpu.VMEM_SHARED((_NPAD, d), jnp.float32),
            pltpu.VMEM((_CH,), jnp.int32),
            pltpu.VMEM((_CH, d), jnp.float32),
            pltpu.VMEM((_ZR, d), jnp.float32),
        ],
    )
    return run(dst, ones, zeros)


def _scatter_body(epw, nchunk, rpt, hp_hbm, src_hbm, dst_hbm, zeros_hbm,
                  out_hbm, acc_sp, idx_s, idx_d, rows, zb, sem):
    c = lax.axis_index("c")
    s = lax.axis_index("s")
    wid = c * _NS + s
    pltpu.sync_copy(zeros_hbm, zb)
    nz = rpt // _ZR
    for k in range(nz):
        pltpu.sync_copy(zb, acc_sp.at[pl.ds(s * rpt + k * _ZR, _ZR)])
    plsc.subcore_barrier()

    def body(i, carry):
        off = wid * epw + i * _CH
        pltpu.sync_copy(src_hbm.at[pl.ds(off, _CH)], idx_s)
        pltpu.sync_copy(dst_hbm.at[pl.ds(off, _CH)], idx_d)
        pltpu.async_copy(hp_hbm.at[idx_s], rows, sem).wait()
        pltpu.sync_copy(rows, acc_sp.at[idx_d], add=True)
        return carry

    lax.fori_loop(0, nchunk, body, 0)
    plsc.subcore_barrier()
    for k in range(nz):
        r0 = s * rpt + k * _ZR
        pltpu.sync_copy(acc_sp.at[pl.ds(r0, _ZR)], zb)
        pltpu.sync_copy(zb, out_hbm.at[c, pl.ds(r0, _ZR)])


def _sc_scatter(hp, src, dst):
    n, d = hp.shape
    e = src.shape[0]
    epw = e // (_NC * _NS)
    nchunk = epw // _CH
    rpt = _NPAD // _NS
    zeros = jnp.zeros((_ZR, d), jnp.float32)
    mesh = plsc.VectorSubcoreMesh(core_axis_name="c", subcore_axis_name="s",
                                  num_cores=_NC, num_subcores=_NS)
    run = pl.kernel(
        functools.partial(_scatter_body, epw, nchunk, rpt),
        out_type=jax.ShapeDtypeStruct((_NC, _NPAD, d), jnp.float32),
        mesh=mesh,
        scratch_types=[
            pltpu.VMEM_SHARED((_NPAD, d), jnp.float32),
            pltpu.VMEM((_CH,), jnp.int32),
            pltpu.VMEM((_CH,), jnp.int32),
            pltpu.VMEM((_CH, d), jnp.float32),
            pltpu.VMEM((_ZR, d), jnp.float32),
            pltpu.SemaphoreType.DMA,
        ],
    )
    return run(hp, src, dst, zeros)


def _dinv_block(dp):
    deg = dp[0, :, 0:1] + dp[1, :, 0:1] + 1.0
    return 1.0 / jnp.sqrt(deg)


def _pre_body(x_ref, w_ref, dp_ref, o_ref):
    dinv = _dinv_block(dp_ref[...])
    h = jnp.dot(x_ref[...], w_ref[...], preferred_element_type=jnp.float32)
    o_ref[...] = h * dinv


def _layer_norm(z, g, b):
    mu = jnp.mean(z, axis=-1, keepdims=True)
    d0 = z - mu
    var = jnp.mean(d0 * d0, axis=-1, keepdims=True)
    return d0 / jnp.sqrt(var + 1e-5) * g + b


def _mid_body(acc_ref, hp_ref, dp_ref, b_ref, g_ref, bb_ref, w_ref, o_ref):
    dinv = _dinv_block(dp_ref[...])
    z = (acc_ref[0] + acc_ref[1] + hp_ref[...]) * dinv + b_ref[...]
    t = jnp.maximum(_layer_norm(z, g_ref[...], bb_ref[...]), 0.0)
    h2 = jnp.dot(t, w_ref[...], preferred_element_type=jnp.float32)
    o_ref[...] = h2 * dinv


def _final_body(acc_ref, hp_ref, dp_ref, b_ref, g_ref, bb_ref, o_ref):
    dinv = _dinv_block(dp_ref[...])
    z = (acc_ref[0] + acc_ref[1] + hp_ref[...]) * dinv + b_ref[...]
    o_ref[...] = jnp.maximum(_layer_norm(z, g_ref[...], bb_ref[...]), 0.0)


_R = 1000


def _row_spec(d):
    return pl.BlockSpec((_R, d), lambda i: (i, 0))


def _vec_spec(d):
    return pl.BlockSpec((1, d), lambda i: (0, 0))


def _full_spec(d):
    return pl.BlockSpec((d, d), lambda i: (0, 0))


def _acc_spec(d):
    return pl.BlockSpec((_NC, _R, d), lambda i: (0, i, 0))


def _tc_pre(x, w, dp):
    n, d = x.shape
    return pl.pallas_call(
        _pre_body,
        grid=(n // _R,),
        in_specs=[_row_spec(d), _full_spec(d), _acc_spec(d)],
        out_specs=_row_spec(d),
        out_shape=jax.ShapeDtypeStruct((n, d), jnp.float32),
    )(x, w, dp)


def _tc_mid(acc, hp, dp, b, g, bb, w):
    n, d = hp.shape
    return pl.pallas_call(
        _mid_body,
        grid=(n // _R,),
        in_specs=[_acc_spec(d), _row_spec(d), _acc_spec(d), _vec_spec(d),
                  _vec_spec(d), _vec_spec(d), _full_spec(d)],
        out_specs=_row_spec(d),
        out_shape=jax.ShapeDtypeStruct((n, d), jnp.float32),
    )(acc, hp, dp, b.reshape(1, d), g.reshape(1, d), bb.reshape(1, d), w)


def _tc_final(acc, hp, dp, b, g, bb):
    n, d = hp.shape
    return pl.pallas_call(
        _final_body,
        grid=(n // _R,),
        in_specs=[_acc_spec(d), _row_spec(d), _acc_spec(d), _vec_spec(d),
                  _vec_spec(d), _vec_spec(d)],
        out_specs=_row_spec(d),
        out_shape=jax.ShapeDtypeStruct((n, d), jnp.float32),
    )(acc, hp, dp, b.reshape(1, d), g.reshape(1, d), bb.reshape(1, d))


def kernel(x, edge_index, W1, b1, ln1_w, ln1_b, W2, b2, ln2_w, ln2_b):
    n, d = x.shape
    src = edge_index[0]
    dst = edge_index[1]

    deg_parts = _sc_degree(dst, n)
    hp1 = _tc_pre(x, W1, deg_parts)
    acc1 = _sc_scatter(hp1, src, dst)
    hp2 = _tc_mid(acc1, hp1, deg_parts, b1, ln1_w, ln1_b, W2)
    acc2 = _sc_scatter(hp2, src, dst)
    return _tc_final(acc2, hp2, deg_parts, b2, ln2_w, ln2_b)


# speedup vs baseline: 22.8383x; 1.8183x over previous
import functools

import jax
import jax.numpy as jnp
from jax import lax
from jax.experimental import pallas as pl
from jax.experimental.pallas import tpu as pltpu
from jax.experimental.pallas import tpu_sc as plsc

_NC = 2
_NS = 16
_NW = _NC * _NS
_EPW = 10240
_CH = 40
_NCH = _EPW // _CH
_DEP = 4
_DCH = 64
_NPAD = 10240
_ZR = 40


def _pad_edges(src, dst, n):
    e = src.shape[0]
    pad = _NW * _EPW - e
    k = jnp.arange(pad, dtype=jnp.int32)
    src_pad = k % n
    dst_pad = n + k % (_NPAD - n)
    srcp = jnp.concatenate([src, src_pad]).reshape(_NW, _NCH, 1, _CH)
    dstp = jnp.concatenate([dst, dst_pad])
    sd = jnp.concatenate(
        [srcp, dstp.reshape(_NW, _NCH, 1, _CH)], axis=2)
    dstr_dg = dstp.reshape(_NW, -1, _DCH)
    return sd, dstr_dg


def _zero_acc(acc_sp, zeros_hbm, buf, sem, s, rpt):
    pltpu.sync_copy(zeros_hbm, buf)
    nz = rpt // _ZR
    for k in range(nz):
        pltpu.async_copy(buf, acc_sp.at[pl.ds(s * rpt + k * _ZR, _ZR)], sem)
    for k in range(nz):
        pltpu.make_async_copy(
            buf, acc_sp.at[pl.ds(0, _ZR)], sem).wait()


def _dump_acc(acc_sp, out_hbm, bufs, sems, c, s, rpt):
    nb = len(bufs)
    nz = rpt // _ZR
    for k in range(nz):
        b = k % nb
        r0 = s * rpt + k * _ZR
        if k >= nb:
            pltpu.make_async_copy(
                bufs[b], out_hbm.at[c, pl.ds(0, _ZR)], sems[b]).wait()
        pltpu.sync_copy(acc_sp.at[pl.ds(r0, _ZR)], bufs[b])
        pltpu.async_copy(bufs[b], out_hbm.at[c, pl.ds(r0, _ZR)], sems[b])
    for k in range(nb):
        pltpu.make_async_copy(
            bufs[k], out_hbm.at[c, pl.ds(0, _ZR)], sems[k]).wait()


def _degree_body(nch, rpt, dstr_hbm, ones_hbm, zeros_hbm, out_hbm,
                 deg_sp, idxd, ones_v, buf0, buf1, sem, sz, sd0, sd1):
    c = lax.axis_index("c")
    s = lax.axis_index("s")
    wid = c * _NS + s
    pltpu.sync_copy(dstr_hbm.at[wid], idxd)
    pltpu.sync_copy(ones_hbm, ones_v)
    _zero_acc(deg_sp, zeros_hbm, buf0, sz, s, rpt)
    plsc.subcore_barrier()

    grp = 8

    def body(j, carry):
        for b in range(grp):
            pltpu.async_copy(ones_v, deg_sp.at[idxd.at[j * grp + b]], sem,
                             add=True)
        for b in range(grp):
            pltpu.make_async_copy(ones_v, deg_sp.at[idxd.at[0]], sem).wait()
        return carry

    lax.fori_loop(0, nch // grp, body, 0)
    plsc.subcore_barrier()
    _dump_acc(deg_sp, out_hbm, [buf0, buf1], [sd0, sd1], c, s, rpt)


def _sc_degree(dstr):
    d = 128
    nch = dstr.shape[1]
    rpt = _NPAD // _NS
    ones = jnp.ones((_DCH, d), jnp.float32)
    zeros = jnp.zeros((_ZR, d), jnp.float32)
    mesh = plsc.VectorSubcoreMesh(core_axis_name="c", subcore_axis_name="s",
                                  num_cores=_NC, num_subcores=_NS)
    run = pl.kernel(
        functools.partial(_degree_body, nch, rpt),
        out_type=jax.ShapeDtypeStruct((_NC, _NPAD, d), jnp.float32),
        mesh=mesh,
        scratch_types=[
            pltpu.VMEM_SHARED((_NPAD, d), jnp.float32),
            pltpu.VMEM((nch, _DCH), jnp.int32),
            pltpu.VMEM((_DCH, d), jnp.float32),
            pltpu.VMEM((_ZR, d), jnp.float32),
            pltpu.VMEM((_ZR, d), jnp.float32),
            pltpu.SemaphoreType.DMA,
            pltpu.SemaphoreType.DMA,
            pltpu.SemaphoreType.DMA,
            pltpu.SemaphoreType.DMA,
        ],
    )
    return run(dstr, ones, zeros)


def _scatter_body(nch, rpt, hp_hbm, sd_hbm, zeros_hbm, out_hbm, acc_sp,
                  ib0, ib1, ib2, ib3, rw0, rw1, rw2, rw3,
                  si0, si1, si2, si3, sg0, sg1, sg2, sg3,
                  sc0, sc1, sc2, sc3):
    c = lax.axis_index("c")
    s = lax.axis_index("s")
    wid = c * _NS + s
    ibs = (ib0, ib1, ib2, ib3)
    rws = (rw0, rw1, rw2, rw3)
    sis = (si0, si1, si2, si3)
    sgs = (sg0, sg1, sg2, sg3)
    scs = (sc0, sc1, sc2, sc3)

    _zero_acc(acc_sp, zeros_hbm, rw0, sc0, s, rpt)
    plsc.subcore_barrier()

    def li(i, b):
        pltpu.async_copy(sd_hbm.at[wid, i], ibs[b], sis[b])

    def li_wait(b):
        pltpu.make_async_copy(sd_hbm.at[0, 0], ibs[b], sis[b]).wait()

    def gat(b):
        pltpu.async_copy(hp_hbm.at[ibs[b].at[0]], rws[b], sgs[b])

    def gat_wait(b):
        pltpu.make_async_copy(hp_hbm.at[ibs[b].at[0]], rws[b], sgs[b]).wait()

    def sca(b):
        pltpu.async_copy(rws[b], acc_sp.at[ibs[b].at[1]], scs[b], add=True)

    def sca_wait(b):
        pltpu.make_async_copy(rws[b], acc_sp.at[ibs[b].at[1]], scs[b]).wait()

    for b in range(_DEP):
        li(b, b)

    def quad(j, carry):
        i = _DEP * j
        for b in range(_DEP):
            li_wait(b)
            gat(b)
        for b in range(_DEP):
            gat_wait(b)
            sca(b)

        @pl.when(j < nch // _DEP - 1)
        def _():
            for b in range(_DEP):
                sca_wait(b)
                li(i + _DEP + b, b)

        return carry

    lax.fori_loop(0, nch // _DEP, quad, 0)
    for b in range(_DEP):
        sca_wait(b)
    plsc.subcore_barrier()
    _dump_acc(acc_sp, out_hbm, list(rws), list(sgs), c, s, rpt)


def _sc_scatter(hp, sd):
    n, d = hp.shape
    nch = sd.shape[1]
    rpt = _NPAD // _NS
    zeros = jnp.zeros((_ZR, d), jnp.float32)
    mesh = plsc.VectorSubcoreMesh(core_axis_name="c", subcore_axis_name="s",
                                  num_cores=_NC, num_subcores=_NS)
    run = pl.kernel(
        functools.partial(_scatter_body, nch, rpt),
        out_type=jax.ShapeDtypeStruct((_NC, _NPAD, d), jnp.float32),
        mesh=mesh,
        scratch_types=(
            [pltpu.VMEM_SHARED((_NPAD, d), jnp.float32)]
            + [pltpu.VMEM((2, _CH), jnp.int32) for _ in range(_DEP)]
            + [pltpu.VMEM((_CH, d), jnp.float32) for _ in range(_DEP)]
            + [pltpu.SemaphoreType.DMA for _ in range(3 * _DEP)]
        ),
    )
    return run(hp, sd, zeros)


def _dinv_block(dp):
    deg = dp[0, :, 0:1] + dp[1, :, 0:1] + 1.0
    return 1.0 / jnp.sqrt(deg)


def _pre_body(x_ref, w_ref, dp_ref, o_ref):
    dinv = _dinv_block(dp_ref[...])
    h = jnp.dot(x_ref[...], w_ref[...], preferred_element_type=jnp.float32)
    o_ref[...] = h * dinv


def _layer_norm(z, g, b):
    mu = jnp.mean(z, axis=-1, keepdims=True)
    d0 = z - mu
    var = jnp.mean(d0 * d0, axis=-1, keepdims=True)
    return d0 / jnp.sqrt(var + 1e-5) * g + b


def _mid_body(acc_ref, hp_ref, dp_ref, b_ref, g_ref, bb_ref, w_ref, o_ref):
    dinv = _dinv_block(dp_ref[...])
    z = (acc_ref[0] + acc_ref[1] + hp_ref[...]) * dinv + b_ref[...]
    t = jnp.maximum(_layer_norm(z, g_ref[...], bb_ref[...]), 0.0)
    h2 = jnp.dot(t, w_ref[...], preferred_element_type=jnp.float32)
    o_ref[...] = h2 * dinv


def _final_body(acc_ref, hp_ref, dp_ref, b_ref, g_ref, bb_ref, o_ref):
    dinv = _dinv_block(dp_ref[...])
    z = (acc_ref[0] + acc_ref[1] + hp_ref[...]) * dinv + b_ref[...]
    o_ref[...] = jnp.maximum(_layer_norm(z, g_ref[...], bb_ref[...]), 0.0)


_R = 1000


def _row_spec(d):
    return pl.BlockSpec((_R, d), lambda i: (i, 0))


def _vec_spec(d):
    return pl.BlockSpec((1, d), lambda i: (0, 0))


def _full_spec(d):
    return pl.BlockSpec((d, d), lambda i: (0, 0))


def _acc_spec(d):
    return pl.BlockSpec((_NC, _R, d), lambda i: (0, i, 0))


def _tc_pre(x, w, dp):
    n, d = x.shape
    return pl.pallas_call(
        _pre_body,
        grid=(n // _R,),
        in_specs=[_row_spec(d), _full_spec(d), _acc_spec(d)],
        out_specs=_row_spec(d),
        out_shape=jax.ShapeDtypeStruct((n, d), jnp.float32),
    )(x, w, dp)


def _tc_mid(acc, hp, dp, b, g, bb, w):
    n, d = hp.shape
    return pl.pallas_call(
        _mid_body,
        grid=(n // _R,),
        in_specs=[_acc_spec(d), _row_spec(d), _acc_spec(d), _vec_spec(d),
                  _vec_spec(d), _vec_spec(d), _full_spec(d)],
        out_specs=_row_spec(d),
        out_shape=jax.ShapeDtypeStruct((n, d), jnp.float32),
    )(acc, hp, dp, b.reshape(1, d), g.reshape(1, d), bb.reshape(1, d), w)


def _tc_final(acc, hp, dp, b, g, bb):
    n, d = hp.shape
    return pl.pallas_call(
        _final_body,
        grid=(n // _R,),
        in_specs=[_acc_spec(d), _row_spec(d), _acc_spec(d), _vec_spec(d),
                  _vec_spec(d), _vec_spec(d)],
        out_specs=_row_spec(d),
        out_shape=jax.ShapeDtypeStruct((n, d), jnp.float32),
    )(acc, hp, dp, b.reshape(1, d), g.reshape(1, d), bb.reshape(1, d))


def kernel(x, edge_index, W1, b1, ln1_w, ln1_b, W2, b2, ln2_w, ln2_b):
    n, d = x.shape
    sd, dstr_dg = _pad_edges(edge_index[0], edge_index[1], n)

    dp = _sc_degree(dstr_dg)
    hp1 = _tc_pre(x, W1, dp)
    acc1 = _sc_scatter(hp1, sd)
    hp2 = _tc_mid(acc1, hp1, dp, b1, ln1_w, ln1_b, W2)
    acc2 = _sc_scatter(hp2, sd)
    return _tc_final(acc2, hp2, dp, b2, ln2_w, ln2_b)


# speedup vs baseline: 24.3697x; 1.0671x over previous
import functools

import jax
import jax.numpy as jnp
from jax import lax
from jax.experimental import pallas as pl
from jax.experimental.pallas import tpu as pltpu
from jax.experimental.pallas import tpu_sc as plsc

_NC = 2
_NS = 16
_NW = _NC * _NS
_EPW = 10240
_CH = 64
_NCH = _EPW // _CH
_DEP = 4
_DCH = 64
_NPAD = 10240
_ZR = 64


def _pad_edges(src, dst, n):
    e = src.shape[0]
    pad = _NW * _EPW - e
    k = jnp.arange(pad, dtype=jnp.int32)
    src_pad = k % n
    dst_pad = n + k % (_NPAD - n)
    srcp = jnp.concatenate([src, src_pad]).reshape(_NW, _NCH, 1, _CH)
    dstp = jnp.concatenate([dst, dst_pad])
    sd = jnp.concatenate(
        [srcp, dstp.reshape(_NW, _NCH, 1, _CH)], axis=2)
    dstr_dg = dstp.reshape(_NW, -1, _DCH)
    return sd, dstr_dg


def _zero_acc(acc_sp, zeros_hbm, buf, sem, s, rpt):
    pltpu.sync_copy(zeros_hbm, buf)
    nz = rpt // _ZR
    for k in range(nz):
        pltpu.async_copy(buf, acc_sp.at[pl.ds(s * rpt + k * _ZR, _ZR)], sem)
    for k in range(nz):
        pltpu.make_async_copy(
            buf, acc_sp.at[pl.ds(0, _ZR)], sem).wait()


def _dump_acc(acc_sp, out_hbm, bufs, sems, c, s, rpt):
    nb = len(bufs)
    nz = rpt // _ZR
    for k in range(nz):
        b = k % nb
        r0 = s * rpt + k * _ZR
        if k >= nb:
            pltpu.make_async_copy(
                bufs[b], out_hbm.at[c, pl.ds(0, _ZR)], sems[b]).wait()
        pltpu.sync_copy(acc_sp.at[pl.ds(r0, _ZR)], bufs[b])
        pltpu.async_copy(bufs[b], out_hbm.at[c, pl.ds(r0, _ZR)], sems[b])
    for k in range(nb):
        pltpu.make_async_copy(
            bufs[k], out_hbm.at[c, pl.ds(0, _ZR)], sems[k]).wait()


def _degree_body(nch, rpt, dstr_hbm, ones_hbm, zeros_hbm, out_hbm,
                 deg_sp, idxd, ones_v, buf0, buf1, sem, sz, sd0, sd1):
    c = lax.axis_index("c")
    s = lax.axis_index("s")
    wid = c * _NS + s
    pltpu.sync_copy(dstr_hbm.at[wid], idxd)
    pltpu.sync_copy(ones_hbm, ones_v)
    _zero_acc(deg_sp, zeros_hbm, buf0, sz, s, rpt)
    plsc.subcore_barrier()

    grp = 16

    def body(j, carry):
        for b in range(grp):
            pltpu.async_copy(ones_v, deg_sp.at[idxd.at[j * grp + b]], sem,
                             add=True)
        for b in range(grp):
            pltpu.make_async_copy(ones_v, deg_sp.at[idxd.at[0]], sem).wait()
        return carry

    lax.fori_loop(0, nch // grp, body, 0)
    plsc.subcore_barrier()
    _dump_acc(deg_sp, out_hbm, [buf0, buf1], [sd0, sd1], c, s, rpt)


def _sc_degree(dstr):
    d = 128
    nch = dstr.shape[1]
    rpt = _NPAD // _NS
    ones = jnp.ones((_DCH, d), jnp.float32)
    zeros = jnp.zeros((_ZR, d), jnp.float32)
    mesh = plsc.VectorSubcoreMesh(core_axis_name="c", subcore_axis_name="s",
                                  num_cores=_NC, num_subcores=_NS)
    run = pl.kernel(
        functools.partial(_degree_body, nch, rpt),
        out_type=jax.ShapeDtypeStruct((_NC, _NPAD, d), jnp.float32),
        mesh=mesh,
        scratch_types=[
            pltpu.VMEM_SHARED((_NPAD, d), jnp.float32),
            pltpu.VMEM((nch, _DCH), jnp.int32),
            pltpu.VMEM((_DCH, d), jnp.float32),
            pltpu.VMEM((_ZR, d), jnp.float32),
            pltpu.VMEM((_ZR, d), jnp.float32),
            pltpu.SemaphoreType.DMA,
            pltpu.SemaphoreType.DMA,
            pltpu.SemaphoreType.DMA,
            pltpu.SemaphoreType.DMA,
        ],
    )
    return run(dstr, ones, zeros)


def _scatter_body(nch, rpt, hp_hbm, sd_hbm, zeros_hbm, out_hbm, acc_sp,
                  ib0, ib1, ib2, ib3, rw0, rw1, rw2, rw3,
                  si0, si1, si2, si3, sg0, sg1, sg2, sg3,
                  sc0, sc1, sc2, sc3):
    c = lax.axis_index("c")
    s = lax.axis_index("s")
    wid = c * _NS + s
    ibs = (ib0, ib1, ib2, ib3)
    rws = (rw0, rw1, rw2, rw3)
    sis = (si0, si1, si2, si3)
    sgs = (sg0, sg1, sg2, sg3)
    scs = (sc0, sc1, sc2, sc3)

    _zero_acc(acc_sp, zeros_hbm, rw0, sc0, s, rpt)
    plsc.subcore_barrier()

    def li(i, b):
        pltpu.async_copy(sd_hbm.at[wid, i], ibs[b], sis[b])

    def li_wait(b):
        pltpu.make_async_copy(sd_hbm.at[0, 0], ibs[b], sis[b]).wait()

    def gat(b):
        pltpu.async_copy(hp_hbm.at[ibs[b].at[0]], rws[b], sgs[b])

    def gat_wait(b):
        pltpu.make_async_copy(hp_hbm.at[ibs[b].at[0]], rws[b], sgs[b]).wait()

    def sca(b):
        pltpu.async_copy(rws[b], acc_sp.at[ibs[b].at[1]], scs[b], add=True)

    def sca_wait(b):
        pltpu.make_async_copy(rws[b], acc_sp.at[ibs[b].at[1]], scs[b]).wait()

    for b in range(_DEP):
        li(b, b)

    def quad(j, carry):
        i = _DEP * j
        for b in range(_DEP):
            li_wait(b)
            gat(b)
        for b in range(_DEP):
            gat_wait(b)
            sca(b)

        @pl.when(j < nch // _DEP - 1)
        def _():
            for b in range(_DEP):
                sca_wait(b)
                li(i + _DEP + b, b)

        return carry

    lax.fori_loop(0, nch // _DEP, quad, 0)
    for b in range(_DEP):
        sca_wait(b)
    plsc.subcore_barrier()
    _dump_acc(acc_sp, out_hbm, list(rws), list(sgs), c, s, rpt)


def _sc_scatter(hp, sd):
    n, d = hp.shape
    nch = sd.shape[1]
    rpt = _NPAD // _NS
    zeros = jnp.zeros((_ZR, d), jnp.float32)
    mesh = plsc.VectorSubcoreMesh(core_axis_name="c", subcore_axis_name="s",
                                  num_cores=_NC, num_subcores=_NS)
    run = pl.kernel(
        functools.partial(_scatter_body, nch, rpt),
        out_type=jax.ShapeDtypeStruct((_NC, _NPAD, d), jnp.float32),
        mesh=mesh,
        scratch_types=(
            [pltpu.VMEM_SHARED((_NPAD, d), jnp.float32)]
            + [pltpu.VMEM((2, _CH), jnp.int32) for _ in range(_DEP)]
            + [pltpu.VMEM((_CH, d), jnp.float32) for _ in range(_DEP)]
            + [pltpu.SemaphoreType.DMA for _ in range(3 * _DEP)]
        ),
    )
    return run(hp, sd, zeros)


def _dinv_block(dp):
    deg = dp[0, :, 0:1] + dp[1, :, 0:1] + 1.0
    return 1.0 / jnp.sqrt(deg)


def _pre_body(x_ref, w_ref, dp_ref, o_ref):
    dinv = _dinv_block(dp_ref[...])
    h = jnp.dot(x_ref[...], w_ref[...], preferred_element_type=jnp.float32)
    o_ref[...] = h * dinv


def _layer_norm(z, g, b):
    mu = jnp.mean(z, axis=-1, keepdims=True)
    d0 = z - mu
    var = jnp.mean(d0 * d0, axis=-1, keepdims=True)
    return d0 / jnp.sqrt(var + 1e-5) * g + b


def _mid_body(acc_ref, hp_ref, dp_ref, b_ref, g_ref, bb_ref, w_ref, o_ref):
    dinv = _dinv_block(dp_ref[...])
    z = (acc_ref[0] + acc_ref[1] + hp_ref[...]) * dinv + b_ref[...]
    t = jnp.maximum(_layer_norm(z, g_ref[...], bb_ref[...]), 0.0)
    h2 = jnp.dot(t, w_ref[...], preferred_element_type=jnp.float32)
    o_ref[...] = h2 * dinv


def _final_body(acc_ref, hp_ref, dp_ref, b_ref, g_ref, bb_ref, o_ref):
    dinv = _dinv_block(dp_ref[...])
    z = (acc_ref[0] + acc_ref[1] + hp_ref[...]) * dinv + b_ref[...]
    o_ref[...] = jnp.maximum(_layer_norm(z, g_ref[...], bb_ref[...]), 0.0)


_R = 1000


def _row_spec(d):
    return pl.BlockSpec((_R, d), lambda i: (i, 0))


def _vec_spec(d):
    return pl.BlockSpec((1, d), lambda i: (0, 0))


def _full_spec(d):
    return pl.BlockSpec((d, d), lambda i: (0, 0))


def _acc_spec(d):
    return pl.BlockSpec((_NC, _R, d), lambda i: (0, i, 0))


def _tc_pre(x, w, dp):
    n, d = x.shape
    return pl.pallas_call(
        _pre_body,
        grid=(n // _R,),
        in_specs=[_row_spec(d), _full_spec(d), _acc_spec(d)],
        out_specs=_row_spec(d),
        out_shape=jax.ShapeDtypeStruct((n, d), jnp.float32),
    )(x, w, dp)


def _tc_mid(acc, hp, dp, b, g, bb, w):
    n, d = hp.shape
    return pl.pallas_call(
        _mid_body,
        grid=(n // _R,),
        in_specs=[_acc_spec(d), _row_spec(d), _acc_spec(d), _vec_spec(d),
                  _vec_spec(d), _vec_spec(d), _full_spec(d)],
        out_specs=_row_spec(d),
        out_shape=jax.ShapeDtypeStruct((n, d), jnp.float32),
    )(acc, hp, dp, b.reshape(1, d), g.reshape(1, d), bb.reshape(1, d), w)


def _tc_final(acc, hp, dp, b, g, bb):
    n, d = hp.shape
    return pl.pallas_call(
        _final_body,
        grid=(n // _R,),
        in_specs=[_acc_spec(d), _row_spec(d), _acc_spec(d), _vec_spec(d),
                  _vec_spec(d), _vec_spec(d)],
        out_specs=_row_spec(d),
        out_shape=jax.ShapeDtypeStruct((n, d), jnp.float32),
    )(acc, hp, dp, b.reshape(1, d), g.reshape(1, d), bb.reshape(1, d))


def kernel(x, edge_index, W1, b1, ln1_w, ln1_b, W2, b2, ln2_w, ln2_b):
    n, d = x.shape
    sd, dstr_dg = _pad_edges(edge_index[0], edge_index[1], n)

    dp = _sc_degree(dstr_dg)
    hp1 = _tc_pre(x, W1, dp)
    acc1 = _sc_scatter(hp1, sd)
    hp2 = _tc_mid(acc1, hp1, dp, b1, ln1_w, ln1_b, W2)
    acc2 = _sc_scatter(hp2, sd)
    return _tc_final(acc2, hp2, dp, b2, ln2_w, ln2_b)


# speedup vs baseline: 24.6495x; 1.0115x over previous
import functools

import jax
import jax.numpy as jnp
from jax import lax
from jax.experimental import pallas as pl
from jax.experimental.pallas import tpu as pltpu
from jax.experimental.pallas import tpu_sc as plsc

_NC = 2
_NS = 16
_NW = _NC * _NS
_EPW = 10240
_CH = 32
_NCH = _EPW // _CH
_DEP = 8
_DCH = 64
_NPAD = 10240
_ZR = _CH


def _pad_edges(src, dst, n):
    e = src.shape[0]
    pad = _NW * _EPW - e
    k = jnp.arange(pad, dtype=jnp.int32)
    src_pad = k % n
    dst_pad = n + k % (_NPAD - n)
    srcp = jnp.concatenate([src, src_pad]).reshape(_NW, _NCH, 1, _CH)
    dstp = jnp.concatenate([dst, dst_pad])
    sd = jnp.concatenate([srcp, dstp.reshape(_NW, _NCH, 1, _CH)], axis=2)
    dstr_dg = dstp.reshape(_NW, -1, _DCH)
    return sd, dstr_dg


def _zero_acc(acc_sp, buf, sem, s, rpt):
    z16 = jnp.zeros((16,), jnp.float32)

    def fill(i, carry):
        buf[i // 8, pl.ds((i % 8) * 16, 16)] = z16
        return carry

    lax.fori_loop(0, _ZR * 8, fill, 0)
    nz = rpt // _ZR
    for k in range(nz):
        pltpu.async_copy(buf, acc_sp.at[pl.ds(s * rpt + k * _ZR, _ZR)], sem)
    for k in range(nz):
        pltpu.make_async_copy(
            buf, acc_sp.at[pl.ds(0, _ZR)], sem).wait()


def _dump_acc(acc_sp, out_hbm, bufs, sems, c, s, rpt):
    nb = len(bufs)
    nz = rpt // _ZR
    for k in range(nz):
        b = k % nb
        r0 = s * rpt + k * _ZR
        if k >= nb:
            pltpu.make_async_copy(
                bufs[b], out_hbm.at[c, pl.ds(0, _ZR)], sems[b]).wait()
        pltpu.sync_copy(acc_sp.at[pl.ds(r0, _ZR)], bufs[b])
        pltpu.async_copy(bufs[b], out_hbm.at[c, pl.ds(r0, _ZR)], sems[b])
    for k in range(nb):
        pltpu.make_async_copy(
            bufs[k], out_hbm.at[c, pl.ds(0, _ZR)], sems[k]).wait()


def _degree_body(nch, rpt, dstr_hbm, ones_hbm, out_hbm,
                 deg_sp, idxd, ones_v, buf0, buf1, sem, sz, sd0, sd1):
    c = lax.axis_index("c")
    s = lax.axis_index("s")
    wid = c * _NS + s
    pltpu.sync_copy(dstr_hbm.at[wid], idxd)
    pltpu.sync_copy(ones_hbm, ones_v)
    _zero_acc(deg_sp, buf0, sz, s, rpt)
    plsc.subcore_barrier()

    grp = 16

    def body(j, carry):
        for b in range(grp):
            pltpu.async_copy(ones_v, deg_sp.at[idxd.at[j * grp + b]], sem,
                             add=True)
        for b in range(grp):
            pltpu.make_async_copy(ones_v, deg_sp.at[idxd.at[0]], sem).wait()
        return carry

    lax.fori_loop(0, nch // grp, body, 0)
    plsc.subcore_barrier()
    _dump_acc(deg_sp, out_hbm, [buf0, buf1], [sd0, sd1], c, s, rpt)


def _sc_degree(dstr):
    d = 128
    nch = dstr.shape[1]
    rpt = _NPAD // _NS
    ones = jnp.ones((_DCH, d), jnp.float32)
    mesh = plsc.VectorSubcoreMesh(core_axis_name="c", subcore_axis_name="s",
                                  num_cores=_NC, num_subcores=_NS)
    run = pl.kernel(
        functools.partial(_degree_body, nch, rpt),
        out_type=jax.ShapeDtypeStruct((_NC, _NPAD, d), jnp.float32),
        mesh=mesh,
        scratch_types=[
            pltpu.VMEM_SHARED((_NPAD, d), jnp.float32),
            pltpu.VMEM((nch, _DCH), jnp.int32),
            pltpu.VMEM((_DCH, d), jnp.float32),
            pltpu.VMEM((_ZR, d), jnp.float32),
            pltpu.VMEM((_ZR, d), jnp.float32),
            pltpu.SemaphoreType.DMA,
            pltpu.SemaphoreType.DMA,
            pltpu.SemaphoreType.DMA,
            pltpu.SemaphoreType.DMA,
        ],
    )
    return run(dstr, ones)


def _scatter_body(nch, rpt, hp_hbm, sd_hbm, out_hbm, acc_sp, *refs):
    ibs = refs[:_DEP]
    rws = refs[_DEP:2 * _DEP]
    sis = refs[2 * _DEP:3 * _DEP]
    sgs = refs[3 * _DEP:4 * _DEP]
    scs = refs[4 * _DEP:5 * _DEP]
    c = lax.axis_index("c")
    s = lax.axis_index("s")
    wid = c * _NS + s

    _zero_acc(acc_sp, rws[0], scs[0], s, rpt)
    plsc.subcore_barrier()

    def li(i, b):
        pltpu.async_copy(sd_hbm.at[wid, i], ibs[b], sis[b])

    def li_wait(b):
        pltpu.make_async_copy(sd_hbm.at[0, 0], ibs[b], sis[b]).wait()

    def gat(b):
        pltpu.async_copy(hp_hbm.at[ibs[b].at[0]], rws[b], sgs[b])

    def gat_wait(b):
        pltpu.make_async_copy(hp_hbm.at[ibs[b].at[0]], rws[b], sgs[b]).wait()

    def sca(b):
        pltpu.async_copy(rws[b], acc_sp.at[ibs[b].at[1]], scs[b], add=True)

    def sca_wait(b):
        pltpu.make_async_copy(rws[b], acc_sp.at[ibs[b].at[1]], scs[b]).wait()

    for b in range(_DEP):
        li(b, b)

    def oct_(j, carry):
        i = _DEP * j
        for b in range(_DEP):
            li_wait(b)
            gat(b)
        for b in range(_DEP):
            gat_wait(b)
            sca(b)

        @pl.when(j < nch // _DEP - 1)
        def _():
            for b in range(_DEP):
                sca_wait(b)
                li(i + _DEP + b, b)

        return carry

    lax.fori_loop(0, nch // _DEP, oct_, 0)
    for b in range(_DEP):
        sca_wait(b)
    plsc.subcore_barrier()
    _dump_acc(acc_sp, out_hbm, list(rws[:4]), list(sgs[:4]), c, s, rpt)


def _sc_scatter(hp, sd):
    n, d = hp.shape
    nch = sd.shape[1]
    rpt = _NPAD // _NS
    mesh = plsc.VectorSubcoreMesh(core_axis_name="c", subcore_axis_name="s",
                                  num_cores=_NC, num_subcores=_NS)
    run = pl.kernel(
        functools.partial(_scatter_body, nch, rpt),
        out_type=jax.ShapeDtypeStruct((_NC, _NPAD, d), jnp.float32),
        mesh=mesh,
        scratch_types=(
            [pltpu.VMEM_SHARED((_NPAD, d), jnp.float32)]
            + [pltpu.VMEM((2, _CH), jnp.int32) for _ in range(_DEP)]
            + [pltpu.VMEM((_CH, d), jnp.float32) for _ in range(_DEP)]
            + [pltpu.SemaphoreType.DMA for _ in range(3 * _DEP)]
        ),
    )
    return run(hp, sd)


def _dinv_block(dp):
    deg = dp[0, :, 0:1] + dp[1, :, 0:1] + 1.0
    return 1.0 / jnp.sqrt(deg)


def _pre_body(x_ref, w_ref, dp_ref, o_ref):
    dinv = _dinv_block(dp_ref[...])
    h = jnp.dot(x_ref[...], w_ref[...], preferred_element_type=jnp.float32)
    o_ref[...] = h * dinv


def _layer_norm(z, g, b):
    mu = jnp.mean(z, axis=-1, keepdims=True)
    d0 = z - mu
    var = jnp.mean(d0 * d0, axis=-1, keepdims=True)
    return d0 / jnp.sqrt(var + 1e-5) * g + b


def _mid_body(acc_ref, hp_ref, dp_ref, b_ref, g_ref, bb_ref, w_ref, o_ref):
    dinv = _dinv_block(dp_ref[...])
    z = (acc_ref[0] + acc_ref[1] + hp_ref[...]) * dinv + b_ref[...]
    t = jnp.maximum(_layer_norm(z, g_ref[...], bb_ref[...]), 0.0)
    h2 = jnp.dot(t, w_ref[...], preferred_element_type=jnp.float32)
    o_ref[...] = h2 * dinv


def _final_body(acc_ref, hp_ref, dp_ref, b_ref, g_ref, bb_ref, o_ref):
    dinv = _dinv_block(dp_ref[...])
    z = (acc_ref[0] + acc_ref[1] + hp_ref[...]) * dinv + b_ref[...]
    o_ref[...] = jnp.maximum(_layer_norm(z, g_ref[...], bb_ref[...]), 0.0)


_R = 1000


def _row_spec(d):
    return pl.BlockSpec((_R, d), lambda i: (i, 0))


def _vec_spec(d):
    return pl.BlockSpec((1, d), lambda i: (0, 0))


def _full_spec(d):
    return pl.BlockSpec((d, d), lambda i: (0, 0))


def _acc_spec(d):
    return pl.BlockSpec((_NC, _R, d), lambda i: (0, i, 0))


def _tc_pre(x, w, dp):
    n, d = x.shape
    return pl.pallas_call(
        _pre_body,
        grid=(n // _R,),
        in_specs=[_row_spec(d), _full_spec(d), _acc_spec(d)],
        out_specs=_row_spec(d),
        out_shape=jax.ShapeDtypeStruct((n, d), jnp.float32),
    )(x, w, dp)


def _tc_mid(acc, hp, dp, b, g, bb, w):
    n, d = hp.shape
    return pl.pallas_call(
        _mid_body,
        grid=(n // _R,),
        in_specs=[_acc_spec(d), _row_spec(d), _acc_spec(d), _vec_spec(d),
                  _vec_spec(d), _vec_spec(d), _full_spec(d)],
        out_specs=_row_spec(d),
        out_shape=jax.ShapeDtypeStruct((n, d), jnp.float32),
    )(acc, hp, dp, b.reshape(1, d), g.reshape(1, d), bb.reshape(1, d), w)


def _tc_final(acc, hp, dp, b, g, bb):
    n, d = hp.shape
    return pl.pallas_call(
        _final_body,
        grid=(n // _R,),
        in_specs=[_acc_spec(d), _row_spec(d), _acc_spec(d), _vec_spec(d),
                  _vec_spec(d), _vec_spec(d)],
        out_specs=_row_spec(d),
        out_shape=jax.ShapeDtypeStruct((n, d), jnp.float32),
    )(acc, hp, dp, b.reshape(1, d), g.reshape(1, d), bb.reshape(1, d))


def kernel(x, edge_index, W1, b1, ln1_w, ln1_b, W2, b2, ln2_w, ln2_b):
    n, d = x.shape
    sd, dstr_dg = _pad_edges(edge_index[0], edge_index[1], n)

    dp = _sc_degree(dstr_dg)
    hp1 = _tc_pre(x, W1, dp)
    acc1 = _sc_scatter(hp1, sd)
    hp2 = _tc_mid(acc1, hp1, dp, b1, ln1_w, ln1_b, W2)
    acc2 = _sc_scatter(hp2, sd)
    return _tc_final(acc2, hp2, dp, b2, ln2_w, ln2_b)
